# initial kernel scaffold (unmeasured)
import jax
import jax.numpy as jnp
from jax import lax
from jax.experimental import pallas as pl
from jax.experimental.pallas import tpu as pltpu

B, H, D = 16, 16, 64
CHUNK = 128
SCALE = D ** -0.5


def kernel(Q, K, V):
    b, kseq, h, d = K.shape
    nchunks = kseq // CHUNK

    def body(q_ref, k_ref, v_ref, o_ref,
             m_ref, l_ref, acc_ref, comm_ref, stats_ref,
             send_sems, recv_sems):
        i = pl.program_id(0)
        my_x = lax.axis_index("x")
        my_y = lax.axis_index("y")
        my_z = lax.axis_index("z")
        partner = (1 - my_x, my_y, my_z)

        @pl.when(i == 0)
        def _init():
            m_ref[...] = jnp.full((B, H), -jnp.inf, jnp.float32)
            l_ref[...] = jnp.zeros((B, H), jnp.float32)
            acc_ref[...] = jnp.zeros((B, H, D), jnp.float32)
            barrier = pltpu.get_barrier_semaphore()
            pl.semaphore_signal(
                barrier, inc=1,
                device_id=partner, device_id_type=pl.DeviceIdType.MESH,
            )
            pl.semaphore_wait(barrier, 1)

        q = q_ref[:, 0, :, :]
        k = k_ref[...]
        s = jnp.sum(k * q[:, None, :, :], axis=-1) * SCALE
        m_cur = jnp.max(s, axis=1)
        m_old = m_ref[...]
        m_new = jnp.maximum(m_old, m_cur)
        alpha = jnp.exp(m_old - m_new)
        p = jnp.exp(s - m_new[:, None, :])
        l_ref[...] = l_ref[...] * alpha + jnp.sum(p, axis=1)
        v = v_ref[...]
        pv = jnp.sum(p[..., None] * v, axis=1)
        acc_ref[...] = acc_ref[...] * alpha[..., None] + pv
        m_ref[...] = m_new

        @pl.when(i == nchunks - 1)
        def _exchange_and_finish():
            comm_ref[0] = acc_ref[...]
            stats_ref[0, 0] = m_ref[...]
            stats_ref[0, 1] = l_ref[...]
            rdma_acc = pltpu.make_async_remote_copy(
                src_ref=comm_ref.at[0],
                dst_ref=comm_ref.at[1],
                send_sem=send_sems.at[0],
                recv_sem=recv_sems.at[0],
                device_id=partner,
                device_id_type=pl.DeviceIdType.MESH,
            )
            rdma_stats = pltpu.make_async_remote_copy(
                src_ref=stats_ref.at[0],
                dst_ref=stats_ref.at[1],
                send_sem=send_sems.at[1],
                recv_sem=recv_sems.at[1],
                device_id=partner,
                device_id_type=pl.DeviceIdType.MESH,
            )
            rdma_acc.start()
            rdma_stats.start()
            rdma_acc.wait()
            rdma_stats.wait()

            m_mine = m_ref[...]
            l_mine = l_ref[...]
            m_p = stats_ref[1, 0]
            l_p = stats_ref[1, 1]
            m_tot = jnp.maximum(m_mine, m_p)
            a_mine = jnp.exp(m_mine - m_tot)
            a_p = jnp.exp(m_p - m_tot)
            num = acc_ref[...] * a_mine[..., None] + comm_ref[1] * a_p[..., None]
            den = l_mine * a_mine + l_p * a_p
            o = num / den[..., None]
            o_ref[...] = o[:, None, :, :]

    return pl.pallas_call(
        body,
        grid=(nchunks,),
        out_shape=jax.ShapeDtypeStruct((B, 1, H, D), jnp.float32),
        in_specs=[
            pl.BlockSpec((B, 1, H, D), lambda i: (0, 0, 0, 0)),
            pl.BlockSpec((B, CHUNK, H, D), lambda i: (0, i, 0, 0)),
            pl.BlockSpec((B, CHUNK, H, D), lambda i: (0, i, 0, 0)),
        ],
        out_specs=pl.BlockSpec((B, 1, H, D), lambda i: (0, 0, 0, 0)),
        scratch_shapes=[
            pltpu.VMEM((B, H), jnp.float32),
            pltpu.VMEM((B, H), jnp.float32),
            pltpu.VMEM((B, H, D), jnp.float32),
            pltpu.VMEM((2, B, H, D), jnp.float32),
            pltpu.VMEM((2, 2, B, H), jnp.float32),
            pltpu.SemaphoreType.DMA((2,)),
            pltpu.SemaphoreType.DMA((2,)),
        ],
        compiler_params=pltpu.CompilerParams(collective_id=0),
    )(Q, K, V)


# baseline (device time: 311253 ns/iter reference)
import jax
import jax.numpy as jnp
from jax import lax
from jax.experimental import pallas as pl
from jax.experimental.pallas import tpu as pltpu

B, H, D = 16, 16, 64
CHUNK = 64
SCALE = D ** -0.5


def kernel(Q, K, V):
    b, kseq, h, d = K.shape
    nchunks = kseq // CHUNK

    def body(q_ref, k_ref, v_ref, o_ref,
             m_ref, l_ref, acc_ref, comm_ref, stats_ref,
             send_sems, recv_sems):
        i = pl.program_id(0)
        my_x = lax.axis_index("x")
        my_y = lax.axis_index("y")
        my_z = lax.axis_index("z")
        partner = (1 - my_x, my_y, my_z)

        @pl.when(i == 0)
        def _init():
            m_ref[...] = jnp.full((B, H), -jnp.inf, jnp.float32)
            l_ref[...] = jnp.zeros((B, H), jnp.float32)
            acc_ref[...] = jnp.zeros((B, H, D), jnp.float32)
            barrier = pltpu.get_barrier_semaphore()
            pl.semaphore_signal(
                barrier, inc=1,
                device_id=partner, device_id_type=pl.DeviceIdType.MESH,
            )
            pl.semaphore_wait(barrier, 1)

        q = q_ref[:, 0, :, :]
        k = k_ref[...]
        s = jnp.sum(k * q[:, None, :, :], axis=-1) * SCALE
        m_cur = jnp.max(s, axis=1)
        m_old = m_ref[...]
        m_new = jnp.maximum(m_old, m_cur)
        alpha = jnp.exp(m_old - m_new)
        p = jnp.exp(s - m_new[:, None, :])
        l_ref[...] = l_ref[...] * alpha + jnp.sum(p, axis=1)
        v = v_ref[...]
        pv = jnp.sum(p[..., None] * v, axis=1)
        acc_ref[...] = acc_ref[...] * alpha[..., None] + pv
        m_ref[...] = m_new

        @pl.when(i == nchunks - 1)
        def _exchange_and_finish():
            comm_ref[0] = acc_ref[...]
            stats_ref[0, 0] = m_ref[...]
            stats_ref[0, 1] = l_ref[...]
            rdma_acc = pltpu.make_async_remote_copy(
                src_ref=comm_ref.at[0],
                dst_ref=comm_ref.at[1],
                send_sem=send_sems.at[0],
                recv_sem=recv_sems.at[0],
                device_id=partner,
                device_id_type=pl.DeviceIdType.MESH,
            )
            rdma_stats = pltpu.make_async_remote_copy(
                src_ref=stats_ref.at[0],
                dst_ref=stats_ref.at[1],
                send_sem=send_sems.at[1],
                recv_sem=recv_sems.at[1],
                device_id=partner,
                device_id_type=pl.DeviceIdType.MESH,
            )
            rdma_acc.start()
            rdma_stats.start()
            rdma_acc.wait()
            rdma_stats.wait()

            m_mine = m_ref[...]
            l_mine = l_ref[...]
            m_p = stats_ref[1, 0]
            l_p = stats_ref[1, 1]
            m_tot = jnp.maximum(m_mine, m_p)
            a_mine = jnp.exp(m_mine - m_tot)
            a_p = jnp.exp(m_p - m_tot)
            num = acc_ref[...] * a_mine[..., None] + comm_ref[1] * a_p[..., None]
            den = l_mine * a_mine + l_p * a_p
            o = num / den[..., None]
            o_ref[...] = o[:, None, :, :]

    return pl.pallas_call(
        body,
        grid=(nchunks,),
        out_shape=jax.ShapeDtypeStruct((B, 1, H, D), jnp.float32),
        in_specs=[
            pl.BlockSpec((B, 1, H, D), lambda i: (0, 0, 0, 0)),
            pl.BlockSpec((B, CHUNK, H, D), lambda i: (0, i, 0, 0)),
            pl.BlockSpec((B, CHUNK, H, D), lambda i: (0, i, 0, 0)),
        ],
        out_specs=pl.BlockSpec((B, 1, H, D), lambda i: (0, 0, 0, 0)),
        scratch_shapes=[
            pltpu.VMEM((B, H), jnp.float32),
            pltpu.VMEM((B, H), jnp.float32),
            pltpu.VMEM((B, H, D), jnp.float32),
            pltpu.VMEM((2, B, H, D), jnp.float32),
            pltpu.VMEM((2, 2, B, H), jnp.float32),
            pltpu.SemaphoreType.DMA((2,)),
            pltpu.SemaphoreType.DMA((2,)),
        ],
        compiler_params=pltpu.CompilerParams(
            collective_id=0, vmem_limit_bytes=48 * 1024 * 1024
        ),
    )(Q, K, V)


# device time: 188641 ns/iter; 1.6500x vs baseline; 1.6500x over previous
import jax
import jax.numpy as jnp
from jax import lax
from jax.experimental import pallas as pl
from jax.experimental.pallas import tpu as pltpu

B, H, D = 16, 16, 64
HD = H * D
CHUNK = 64
SCALE = D ** -0.5


def kernel(Q, K, V):
    b, kseq, h, d = K.shape
    nchunks = kseq // CHUNK

    def body(q_ref, k_ref, v_ref, o_ref,
             m_ref, l_ref, acc_ref, comm_ref, stats_ref,
             send_sems, recv_sems):
        i = pl.program_id(0)
        my_x = lax.axis_index("x")
        my_y = lax.axis_index("y")
        my_z = lax.axis_index("z")
        partner = (1 - my_x, my_y, my_z)

        head_of = jax.lax.broadcasted_iota(jnp.int32, (HD, H), 0) // D
        col = jax.lax.broadcasted_iota(jnp.int32, (HD, H), 1)
        R = (head_of == col).astype(jnp.bfloat16)
        E = R.T

        @pl.when(i == 0)
        def _init():
            m_ref[...] = jnp.full((B, H), -jnp.inf, jnp.float32)
            l_ref[...] = jnp.zeros((B, H), jnp.float32)
            acc_ref[...] = jnp.zeros((B, HD), jnp.float32)
            barrier = pltpu.get_barrier_semaphore()
            pl.semaphore_signal(
                barrier, inc=1,
                device_id=partner, device_id_type=pl.DeviceIdType.MESH,
            )
            pl.semaphore_wait(barrier, 1)

        q = q_ref[...].astype(jnp.bfloat16)
        k = k_ref[...].astype(jnp.bfloat16)
        prod = k * q[:, None, :]
        s = lax.dot_general(
            prod.reshape(B * CHUNK, HD), R,
            dimension_numbers=(((1,), (0,)), ((), ())),
            preferred_element_type=jnp.float32,
        ).reshape(B, CHUNK, H) * SCALE
        m_cur = jnp.max(s, axis=1)
        m_old = m_ref[...]
        m_new = jnp.maximum(m_old, m_cur)
        alpha = jnp.exp(m_old - m_new)
        p = jnp.exp(s - m_new[:, None, :])
        l_ref[...] = l_ref[...] * alpha + jnp.sum(p, axis=1)
        pb = lax.dot_general(
            p.reshape(B * CHUNK, H).astype(jnp.bfloat16), E,
            dimension_numbers=(((1,), (0,)), ((), ())),
            preferred_element_type=jnp.float32,
        ).reshape(B, CHUNK, HD)
        pv = jnp.sum(pb * v_ref[...], axis=1)
        alpha_hd = lax.dot_general(
            alpha.astype(jnp.bfloat16), E,
            dimension_numbers=(((1,), (0,)), ((), ())),
            preferred_element_type=jnp.float32,
        )
        acc_ref[...] = acc_ref[...] * alpha_hd + pv
        m_ref[...] = m_new

        @pl.when(i == nchunks - 1)
        def _exchange_and_finish():
            comm_ref[0] = acc_ref[...]
            stats_ref[0, 0] = m_ref[...]
            stats_ref[0, 1] = l_ref[...]
            rdma_acc = pltpu.make_async_remote_copy(
                src_ref=comm_ref.at[0],
                dst_ref=comm_ref.at[1],
                send_sem=send_sems.at[0],
                recv_sem=recv_sems.at[0],
                device_id=partner,
                device_id_type=pl.DeviceIdType.MESH,
            )
            rdma_stats = pltpu.make_async_remote_copy(
                src_ref=stats_ref.at[0],
                dst_ref=stats_ref.at[1],
                send_sem=send_sems.at[1],
                recv_sem=recv_sems.at[1],
                device_id=partner,
                device_id_type=pl.DeviceIdType.MESH,
            )
            rdma_acc.start()
            rdma_stats.start()
            rdma_acc.wait()
            rdma_stats.wait()

            m_mine = m_ref[...]
            l_mine = l_ref[...]
            m_p = stats_ref[1, 0]
            l_p = stats_ref[1, 1]
            m_tot = jnp.maximum(m_mine, m_p)
            a_mine = jnp.exp(m_mine - m_tot)
            a_p = jnp.exp(m_p - m_tot)
            den = l_mine * a_mine + l_p * a_p

            def expand(x):
                return lax.dot_general(
                    x.astype(jnp.bfloat16), E,
                    dimension_numbers=(((1,), (0,)), ((), ())),
                    preferred_element_type=jnp.float32,
                )

            num = acc_ref[...] * expand(a_mine) + comm_ref[1] * expand(a_p)
            o_ref[...] = num / expand(den)

    out = pl.pallas_call(
        body,
        grid=(nchunks,),
        out_shape=jax.ShapeDtypeStruct((B, HD), jnp.float32),
        in_specs=[
            pl.BlockSpec((B, HD), lambda i: (0, 0)),
            pl.BlockSpec((B, CHUNK, HD), lambda i: (0, i, 0)),
            pl.BlockSpec((B, CHUNK, HD), lambda i: (0, i, 0)),
        ],
        out_specs=pl.BlockSpec((B, HD), lambda i: (0, 0)),
        scratch_shapes=[
            pltpu.VMEM((B, H), jnp.float32),
            pltpu.VMEM((B, H), jnp.float32),
            pltpu.VMEM((B, HD), jnp.float32),
            pltpu.VMEM((2, B, HD), jnp.float32),
            pltpu.VMEM((2, 2, B, H), jnp.float32),
            pltpu.SemaphoreType.DMA((2,)),
            pltpu.SemaphoreType.DMA((2,)),
        ],
        compiler_params=pltpu.CompilerParams(
            collective_id=0, vmem_limit_bytes=48 * 1024 * 1024
        ),
    )(Q.reshape(B, HD), K.reshape(b, kseq, HD), V.reshape(b, kseq, HD))
    return out.reshape(B, 1, H, D)


# device time: 183350 ns/iter; 1.6976x vs baseline; 1.0289x over previous
import jax
import jax.numpy as jnp
from jax import lax
from jax.experimental import pallas as pl
from jax.experimental.pallas import tpu as pltpu

B, H, D = 16, 16, 64
HD = H * D
CHUNK = 128
SCALE = D ** -0.5


def kernel(Q, K, V):
    b, kseq, h, d = K.shape
    nchunks = kseq // CHUNK

    def body(q_ref, k_ref, v_ref, o_ref,
             m_ref, l_ref, acc_ref, qmat_ref, comm_ref, stats_ref,
             send_sems, recv_sems):
        i = pl.program_id(0)
        my_x = lax.axis_index("x")
        my_y = lax.axis_index("y")
        my_z = lax.axis_index("z")
        partner = (1 - my_x, my_y, my_z)

        row = jax.lax.broadcasted_iota(jnp.int32, (H, HD), 0)
        head_of = jax.lax.broadcasted_iota(jnp.int32, (H, HD), 1) // D
        E = (row == head_of).astype(jnp.float32)

        @pl.when(i == 0)
        def _init():
            m_ref[...] = jnp.full((B, H), -jnp.inf, jnp.float32)
            l_ref[...] = jnp.zeros((B, H), jnp.float32)
            acc_ref[...] = jnp.zeros((B, HD), jnp.float32)
            hd_head = jax.lax.broadcasted_iota(jnp.int32, (HD, H), 0) // D
            hd_col = jax.lax.broadcasted_iota(jnp.int32, (HD, H), 1)
            R = (hd_head == hd_col).astype(jnp.float32)
            qmat_ref[...] = q_ref[...][:, :, None] * R[None, :, :] * SCALE
            barrier = pltpu.get_barrier_semaphore()
            pl.semaphore_signal(
                barrier, inc=1,
                device_id=partner, device_id_type=pl.DeviceIdType.MESH,
            )
            pl.semaphore_wait(barrier, 1)

        s = lax.dot_general(
            k_ref[...], qmat_ref[...],
            dimension_numbers=(((2,), (1,)), ((0,), (0,))),
            preferred_element_type=jnp.float32,
        )
        m_cur = jnp.max(s, axis=1)
        m_old = m_ref[...]
        m_new = jnp.maximum(m_old, m_cur)
        alpha = jnp.exp(m_old - m_new)
        p = jnp.exp(s - m_new[:, None, :])
        l_ref[...] = l_ref[...] * alpha + jnp.sum(p, axis=1)
        o_full = lax.dot_general(
            p, v_ref[...],
            dimension_numbers=(((1,), (1,)), ((0,), (0,))),
            preferred_element_type=jnp.float32,
        )
        pv = jnp.sum(o_full * E[None, :, :], axis=1)
        alpha_hd = lax.dot_general(
            alpha, E,
            dimension_numbers=(((1,), (0,)), ((), ())),
            preferred_element_type=jnp.float32,
        )
        acc_ref[...] = acc_ref[...] * alpha_hd + pv
        m_ref[...] = m_new

        @pl.when(i == nchunks - 1)
        def _exchange_and_finish():
            comm_ref[0] = acc_ref[...]
            stats_ref[0, 0] = m_ref[...]
            stats_ref[0, 1] = l_ref[...]
            rdma_acc = pltpu.make_async_remote_copy(
                src_ref=comm_ref.at[0],
                dst_ref=comm_ref.at[1],
                send_sem=send_sems.at[0],
                recv_sem=recv_sems.at[0],
                device_id=partner,
                device_id_type=pl.DeviceIdType.MESH,
            )
            rdma_stats = pltpu.make_async_remote_copy(
                src_ref=stats_ref.at[0],
                dst_ref=stats_ref.at[1],
                send_sem=send_sems.at[1],
                recv_sem=recv_sems.at[1],
                device_id=partner,
                device_id_type=pl.DeviceIdType.MESH,
            )
            rdma_acc.start()
            rdma_stats.start()
            rdma_acc.wait()
            rdma_stats.wait()

            m_mine = m_ref[...]
            l_mine = l_ref[...]
            m_p = stats_ref[1, 0]
            l_p = stats_ref[1, 1]
            m_tot = jnp.maximum(m_mine, m_p)
            a_mine = jnp.exp(m_mine - m_tot)
            a_p = jnp.exp(m_p - m_tot)
            den = l_mine * a_mine + l_p * a_p

            def expand(x):
                return lax.dot_general(
                    x, E,
                    dimension_numbers=(((1,), (0,)), ((), ())),
                    preferred_element_type=jnp.float32,
                )

            num = acc_ref[...] * expand(a_mine) + comm_ref[1] * expand(a_p)
            o_ref[...] = num / expand(den)

    out = pl.pallas_call(
        body,
        grid=(nchunks,),
        out_shape=jax.ShapeDtypeStruct((B, HD), jnp.float32),
        in_specs=[
            pl.BlockSpec((B, HD), lambda i: (0, 0)),
            pl.BlockSpec((B, CHUNK, HD), lambda i: (0, i, 0)),
            pl.BlockSpec((B, CHUNK, HD), lambda i: (0, i, 0)),
        ],
        out_specs=pl.BlockSpec((B, HD), lambda i: (0, 0)),
        scratch_shapes=[
            pltpu.VMEM((B, H), jnp.float32),
            pltpu.VMEM((B, H), jnp.float32),
            pltpu.VMEM((B, HD), jnp.float32),
            pltpu.VMEM((B, HD, H), jnp.float32),
            pltpu.VMEM((2, B, HD), jnp.float32),
            pltpu.VMEM((2, 2, B, H), jnp.float32),
            pltpu.SemaphoreType.DMA((2,)),
            pltpu.SemaphoreType.DMA((2,)),
        ],
        compiler_params=pltpu.CompilerParams(
            collective_id=0, vmem_limit_bytes=56 * 1024 * 1024
        ),
    )(Q.reshape(B, HD), K.reshape(b, kseq, HD), V.reshape(b, kseq, HD))
    return out.reshape(B, 1, H, D)


# device time: 183229 ns/iter; 1.6987x vs baseline; 1.0007x over previous
import jax
import jax.numpy as jnp
from jax import lax
from jax.experimental import pallas as pl
from jax.experimental.pallas import tpu as pltpu

B, H, D = 16, 16, 64
HD = H * D
CHUNK = 128
SCALE = D ** -0.5


def kernel(Q, K, V):
    b, kseq, h, d = K.shape
    nchunks = kseq // CHUNK

    def body(q_ref, k_ref, v_ref, o_ref,
             m_ref, l_ref, acc_ref, qmat_ref, comm_ref, stats_ref,
             send_sems, recv_sems):
        i = pl.program_id(0)
        my_x = lax.axis_index("x")
        my_y = lax.axis_index("y")
        my_z = lax.axis_index("z")
        partner = (1 - my_x, my_y, my_z)

        row = jax.lax.broadcasted_iota(jnp.int32, (H, HD), 0)
        head_of = jax.lax.broadcasted_iota(jnp.int32, (H, HD), 1) // D
        E = (row == head_of).astype(jnp.float32)

        @pl.when(i == 0)
        def _init():
            m_ref[...] = jnp.full((B, H), -jnp.inf, jnp.float32)
            l_ref[...] = jnp.zeros((B, H), jnp.float32)
            acc_ref[...] = jnp.zeros((B, HD), jnp.float32)
            hd_head = jax.lax.broadcasted_iota(jnp.int32, (HD, H), 0) // D
            hd_col = jax.lax.broadcasted_iota(jnp.int32, (HD, H), 1)
            R = (hd_head == hd_col).astype(jnp.float32)
            qmat_ref[...] = (
                q_ref[...][:, :, None] * R[None, :, :] * SCALE
            ).astype(jnp.bfloat16)
            barrier = pltpu.get_barrier_semaphore()
            pl.semaphore_signal(
                barrier, inc=1,
                device_id=partner, device_id_type=pl.DeviceIdType.MESH,
            )
            pl.semaphore_wait(barrier, 1)

        s = lax.dot_general(
            k_ref[...].astype(jnp.bfloat16), qmat_ref[...],
            dimension_numbers=(((2,), (1,)), ((0,), (0,))),
            preferred_element_type=jnp.float32,
        )
        m_cur = jnp.max(s, axis=1)
        m_old = m_ref[...]
        m_new = jnp.maximum(m_old, m_cur)
        alpha = jnp.exp(m_old - m_new)
        p = jnp.exp(s - m_new[:, None, :])
        l_ref[...] = l_ref[...] * alpha + jnp.sum(p, axis=1)
        o_full = lax.dot_general(
            p.astype(jnp.bfloat16), v_ref[...].astype(jnp.bfloat16),
            dimension_numbers=(((1,), (1,)), ((0,), (0,))),
            preferred_element_type=jnp.float32,
        )
        pv = jnp.sum(o_full * E[None, :, :], axis=1)
        alpha_hd = lax.dot_general(
            alpha, E,
            dimension_numbers=(((1,), (0,)), ((), ())),
            preferred_element_type=jnp.float32,
        )
        acc_ref[...] = acc_ref[...] * alpha_hd + pv
        m_ref[...] = m_new

        @pl.when(i == nchunks - 1)
        def _exchange_and_finish():
            comm_ref[0] = acc_ref[...]
            stats_ref[0, 0] = m_ref[...]
            stats_ref[0, 1] = l_ref[...]
            rdma_acc = pltpu.make_async_remote_copy(
                src_ref=comm_ref.at[0],
                dst_ref=comm_ref.at[1],
                send_sem=send_sems.at[0],
                recv_sem=recv_sems.at[0],
                device_id=partner,
                device_id_type=pl.DeviceIdType.MESH,
            )
            rdma_stats = pltpu.make_async_remote_copy(
                src_ref=stats_ref.at[0],
                dst_ref=stats_ref.at[1],
                send_sem=send_sems.at[1],
                recv_sem=recv_sems.at[1],
                device_id=partner,
                device_id_type=pl.DeviceIdType.MESH,
            )
            rdma_acc.start()
            rdma_stats.start()
            rdma_acc.wait()
            rdma_stats.wait()

            m_mine = m_ref[...]
            l_mine = l_ref[...]
            m_p = stats_ref[1, 0]
            l_p = stats_ref[1, 1]
            m_tot = jnp.maximum(m_mine, m_p)
            a_mine = jnp.exp(m_mine - m_tot)
            a_p = jnp.exp(m_p - m_tot)
            den = l_mine * a_mine + l_p * a_p

            def expand(x):
                return lax.dot_general(
                    x, E,
                    dimension_numbers=(((1,), (0,)), ((), ())),
                    preferred_element_type=jnp.float32,
                )

            num = acc_ref[...] * expand(a_mine) + comm_ref[1] * expand(a_p)
            o_ref[...] = num / expand(den)

    out = pl.pallas_call(
        body,
        grid=(nchunks,),
        out_shape=jax.ShapeDtypeStruct((B, HD), jnp.float32),
        in_specs=[
            pl.BlockSpec((B, HD), lambda i: (0, 0)),
            pl.BlockSpec((B, CHUNK, HD), lambda i: (0, i, 0)),
            pl.BlockSpec((B, CHUNK, HD), lambda i: (0, i, 0)),
        ],
        out_specs=pl.BlockSpec((B, HD), lambda i: (0, 0)),
        scratch_shapes=[
            pltpu.VMEM((B, H), jnp.float32),
            pltpu.VMEM((B, H), jnp.float32),
            pltpu.VMEM((B, HD), jnp.float32),
            pltpu.VMEM((B, HD, H), jnp.bfloat16),
            pltpu.VMEM((2, B, HD), jnp.float32),
            pltpu.VMEM((2, 2, B, H), jnp.float32),
            pltpu.SemaphoreType.DMA((2,)),
            pltpu.SemaphoreType.DMA((2,)),
        ],
        compiler_params=pltpu.CompilerParams(
            collective_id=0, vmem_limit_bytes=56 * 1024 * 1024
        ),
    )(Q.reshape(B, HD), K.reshape(b, kseq, HD), V.reshape(b, kseq, HD))
    return out.reshape(B, 1, H, D)


# device time: 53188 ns/iter; 5.8519x vs baseline; 3.4449x over previous
import jax
import jax.numpy as jnp
from jax import lax
from jax.experimental import pallas as pl
from jax.experimental.pallas import tpu as pltpu

B, H, D = 16, 16, 64
HD = H * D
CHUNK = 128
SCALE = D ** -0.5


def kernel(Q, K, V):
    b, kseq, h, d = K.shape
    nchunks = kseq // CHUNK

    def body(q_ref, k_ref, v_ref, o_ref,
             m_ref, l_ref, acc_ref, qs_ref, comm_ref, stats_ref,
             send_sems, recv_sems):
        i = pl.program_id(0)
        my_x = lax.axis_index("x")
        my_y = lax.axis_index("y")
        my_z = lax.axis_index("z")
        partner = (1 - my_x, my_y, my_z)

        row = jax.lax.broadcasted_iota(jnp.int32, (H, HD), 0)
        head_of = jax.lax.broadcasted_iota(jnp.int32, (H, HD), 1) // D
        E = (row == head_of).astype(jnp.float32)

        @pl.when(i == 0)
        def _init():
            m_ref[...] = jnp.full((B, H), -jnp.inf, jnp.float32)
            l_ref[...] = jnp.zeros((B, H), jnp.float32)
            acc_ref[...] = jnp.zeros((B, HD), jnp.float32)
            qs_ref[...] = q_ref[:, 0, :, :][..., None] * SCALE
            barrier = pltpu.get_barrier_semaphore()
            pl.semaphore_signal(
                barrier, inc=1,
                device_id=partner, device_id_type=pl.DeviceIdType.MESH,
            )
            pl.semaphore_wait(barrier, 1)

        s = jnp.sum(k_ref[...] * qs_ref[...], axis=2)
        m_cur = jnp.max(s, axis=-1)
        m_old = m_ref[...]
        m_new = jnp.maximum(m_old, m_cur)
        alpha = jnp.exp(m_old - m_new)
        p = jnp.exp(s - m_new[..., None])
        l_ref[...] = l_ref[...] * alpha + jnp.sum(p, axis=-1)
        m_ref[...] = m_new
        vt = v_ref[...].reshape(B, HD, CHUNK)
        o_full = lax.dot_general(
            p.astype(jnp.bfloat16), vt.astype(jnp.bfloat16),
            dimension_numbers=(((2,), (2,)), ((0,), (0,))),
            preferred_element_type=jnp.float32,
        )
        pv = jnp.sum(o_full * E[None, :, :], axis=1)
        alpha_hd = lax.dot_general(
            alpha, E,
            dimension_numbers=(((1,), (0,)), ((), ())),
            preferred_element_type=jnp.float32,
        )
        acc_ref[...] = acc_ref[...] * alpha_hd + pv

        @pl.when(i == nchunks - 1)
        def _exchange_and_finish():
            comm_ref[0] = acc_ref[...]
            stats_ref[0, 0] = m_ref[...]
            stats_ref[0, 1] = l_ref[...]
            rdma_acc = pltpu.make_async_remote_copy(
                src_ref=comm_ref.at[0],
                dst_ref=comm_ref.at[1],
                send_sem=send_sems.at[0],
                recv_sem=recv_sems.at[0],
                device_id=partner,
                device_id_type=pl.DeviceIdType.MESH,
            )
            rdma_stats = pltpu.make_async_remote_copy(
                src_ref=stats_ref.at[0],
                dst_ref=stats_ref.at[1],
                send_sem=send_sems.at[1],
                recv_sem=recv_sems.at[1],
                device_id=partner,
                device_id_type=pl.DeviceIdType.MESH,
            )
            rdma_acc.start()
            rdma_stats.start()
            rdma_acc.wait()
            rdma_stats.wait()

            m_mine = m_ref[...]
            l_mine = l_ref[...]
            m_p = stats_ref[1, 0]
            l_p = stats_ref[1, 1]
            m_tot = jnp.maximum(m_mine, m_p)
            a_mine = jnp.exp(m_mine - m_tot)
            a_p = jnp.exp(m_p - m_tot)
            den = l_mine * a_mine + l_p * a_p

            def expand(x):
                return lax.dot_general(
                    x, E,
                    dimension_numbers=(((1,), (0,)), ((), ())),
                    preferred_element_type=jnp.float32,
                )

            num = acc_ref[...] * expand(a_mine) + comm_ref[1] * expand(a_p)
            o_ref[...] = num / expand(den)

    kt = jnp.transpose(K, (0, 2, 3, 1))
    vt = jnp.transpose(V, (0, 2, 3, 1))

    out = pl.pallas_call(
        body,
        grid=(nchunks,),
        out_shape=jax.ShapeDtypeStruct((B, HD), jnp.float32),
        in_specs=[
            pl.BlockSpec((B, 1, H, D), lambda i: (0, 0, 0, 0)),
            pl.BlockSpec((B, H, D, CHUNK), lambda i: (0, 0, 0, i)),
            pl.BlockSpec((B, H, D, CHUNK), lambda i: (0, 0, 0, i)),
        ],
        out_specs=pl.BlockSpec((B, HD), lambda i: (0, 0)),
        scratch_shapes=[
            pltpu.VMEM((B, H), jnp.float32),
            pltpu.VMEM((B, H), jnp.float32),
            pltpu.VMEM((B, HD), jnp.float32),
            pltpu.VMEM((B, H, D, 1), jnp.float32),
            pltpu.VMEM((2, B, HD), jnp.float32),
            pltpu.VMEM((2, 2, B, H), jnp.float32),
            pltpu.SemaphoreType.DMA((2,)),
            pltpu.SemaphoreType.DMA((2,)),
        ],
        compiler_params=pltpu.CompilerParams(
            collective_id=0, vmem_limit_bytes=48 * 1024 * 1024
        ),
    )(Q, kt, vt)
    return out.reshape(B, 1, H, D)


# device time: 52167 ns/iter; 5.9665x vs baseline; 1.0196x over previous
import jax
import jax.numpy as jnp
from jax import lax
from jax.experimental import pallas as pl
from jax.experimental.pallas import tpu as pltpu

B, H, D = 16, 16, 64
HD = H * D
BB = 8
CHUNK = 256
SCALE = D ** -0.5


def kernel(Q, K, V):
    b, kseq, h, d = K.shape
    nb = B // BB
    nc = kseq // CHUNK

    def body(q_ref, k_ref, v_ref, o_ref,
             m_ref, l_ref, acc_ref, qs_ref, comm_ref, stats_ref,
             send_sems, recv_sems):
        bi = pl.program_id(0)
        j = pl.program_id(1)
        my_x = lax.axis_index("x")
        my_y = lax.axis_index("y")
        my_z = lax.axis_index("z")
        partner = (1 - my_x, my_y, my_z)
        rows = pl.ds(bi * BB, BB)

        row = jax.lax.broadcasted_iota(jnp.int32, (H, HD), 0)
        head_of = jax.lax.broadcasted_iota(jnp.int32, (H, HD), 1) // D
        E = (row == head_of).astype(jnp.float32)

        @pl.when(jnp.logical_and(bi == 0, j == 0))
        def _first_step():
            qs_ref[...] = q_ref[:, 0, :, :][..., None] * SCALE
            barrier = pltpu.get_barrier_semaphore()
            pl.semaphore_signal(
                barrier, inc=1,
                device_id=partner, device_id_type=pl.DeviceIdType.MESH,
            )
            pl.semaphore_wait(barrier, 1)

        @pl.when(j == 0)
        def _init_block():
            m_ref[rows, :] = jnp.full((BB, H), -jnp.inf, jnp.float32)
            l_ref[rows, :] = jnp.zeros((BB, H), jnp.float32)
            acc_ref[rows, :] = jnp.zeros((BB, HD), jnp.float32)

        qs = qs_ref[rows, :, :, :]
        s = jnp.sum(k_ref[...] * qs, axis=2)
        m_cur = jnp.max(s, axis=-1)
        m_old = m_ref[rows, :]
        m_new = jnp.maximum(m_old, m_cur)
        alpha = jnp.exp(m_old - m_new)
        p = jnp.exp(s - m_new[..., None])
        l_ref[rows, :] = l_ref[rows, :] * alpha + jnp.sum(p, axis=-1)
        m_ref[rows, :] = m_new
        vt = v_ref[...].reshape(BB, HD, CHUNK)
        o_full = lax.dot_general(
            p.astype(jnp.bfloat16), vt.astype(jnp.bfloat16),
            dimension_numbers=(((2,), (2,)), ((0,), (0,))),
            preferred_element_type=jnp.float32,
        )
        pv = jnp.sum(o_full * E[None, :, :], axis=1)
        alpha_hd = lax.dot_general(
            alpha, E,
            dimension_numbers=(((1,), (0,)), ((), ())),
            preferred_element_type=jnp.float32,
        )
        acc_ref[rows, :] = acc_ref[rows, :] * alpha_hd + pv

        @pl.when(jnp.logical_and(bi == nb - 1, j == nc - 1))
        def _exchange_and_finish():
            comm_ref[0] = acc_ref[...]
            stats_ref[0, 0] = m_ref[...]
            stats_ref[0, 1] = l_ref[...]
            rdma_acc = pltpu.make_async_remote_copy(
                src_ref=comm_ref.at[0],
                dst_ref=comm_ref.at[1],
                send_sem=send_sems.at[0],
                recv_sem=recv_sems.at[0],
                device_id=partner,
                device_id_type=pl.DeviceIdType.MESH,
            )
            rdma_stats = pltpu.make_async_remote_copy(
                src_ref=stats_ref.at[0],
                dst_ref=stats_ref.at[1],
                send_sem=send_sems.at[1],
                recv_sem=recv_sems.at[1],
                device_id=partner,
                device_id_type=pl.DeviceIdType.MESH,
            )
            rdma_acc.start()
            rdma_stats.start()
            rdma_acc.wait()
            rdma_stats.wait()

            m_mine = m_ref[...]
            l_mine = l_ref[...]
            m_p = stats_ref[1, 0]
            l_p = stats_ref[1, 1]
            m_tot = jnp.maximum(m_mine, m_p)
            a_mine = jnp.exp(m_mine - m_tot)
            a_p = jnp.exp(m_p - m_tot)
            den = l_mine * a_mine + l_p * a_p

            def expand(x):
                return lax.dot_general(
                    x, E,
                    dimension_numbers=(((1,), (0,)), ((), ())),
                    preferred_element_type=jnp.float32,
                )

            num = acc_ref[...] * expand(a_mine) + comm_ref[1] * expand(a_p)
            o_ref[...] = num / expand(den)

    kt = jnp.transpose(K, (0, 2, 3, 1))
    vt = jnp.transpose(V, (0, 2, 3, 1))

    out = pl.pallas_call(
        body,
        grid=(nb, nc),
        out_shape=jax.ShapeDtypeStruct((B, HD), jnp.float32),
        in_specs=[
            pl.BlockSpec((B, 1, H, D), lambda bi, j: (0, 0, 0, 0)),
            pl.BlockSpec((BB, H, D, CHUNK), lambda bi, j: (bi, 0, 0, j)),
            pl.BlockSpec((BB, H, D, CHUNK), lambda bi, j: (bi, 0, 0, j)),
        ],
        out_specs=pl.BlockSpec((B, HD), lambda bi, j: (0, 0)),
        scratch_shapes=[
            pltpu.VMEM((B, H), jnp.float32),
            pltpu.VMEM((B, H), jnp.float32),
            pltpu.VMEM((B, HD), jnp.float32),
            pltpu.VMEM((B, H, D, 1), jnp.float32),
            pltpu.VMEM((2, B, HD), jnp.float32),
            pltpu.VMEM((2, 2, B, H), jnp.float32),
            pltpu.SemaphoreType.DMA((2,)),
            pltpu.SemaphoreType.DMA((2,)),
        ],
        compiler_params=pltpu.CompilerParams(
            collective_id=0, vmem_limit_bytes=48 * 1024 * 1024
        ),
    )(Q, kt, vt)
    return out.reshape(B, 1, H, D)
